# baseline (speedup 1.0000x reference)
import functools

import jax
import jax.numpy as jnp
from jax import lax
from jax.experimental import pallas as pl
from jax.experimental.pallas import tpu as pltpu
from jax.experimental.pallas import tpu_sc as plsc

N = 10000
D = 128
DH = D // 2
E = 320000

NC = 2
NS = 16
EPW = E // NS
K = 128
NCHUNK = (EPW + K - 1) // K
PAD = NCHUNK * K - EPW
NP = 10240
RPT = NP // NS
ZR = 128
TR = N - (NS - 1) * RPT - 3 * ZR
CW = 16

_mesh = plsc.VectorSubcoreMesh(core_axis_name="c", subcore_axis_name="s")


@functools.partial(
    pl.kernel,
    out_type=jax.ShapeDtypeStruct((N, D), jnp.float32),
    mesh=_mesh,
    compiler_params=pltpu.CompilerParams(use_tc_tiling_on_sc=False),
    scratch_types=[
        pltpu.VMEM((NCHUNK, K), jnp.int32),
        pltpu.VMEM((NCHUNK, K), jnp.int32),
        pltpu.VMEM((K, DH), jnp.float32),
        pltpu.VMEM((K, DH), jnp.float32),
        pltpu.VMEM((ZR, DH), jnp.float32),
        pltpu.VMEM((ZR, CW), jnp.float32),
        pltpu.VMEM((K, CW), jnp.float32),
        pltpu.VMEM_SHARED((NP, DH), jnp.float32),
        pltpu.VMEM_SHARED((NP, CW), jnp.float32),
        pltpu.SemaphoreType.DMA,
        pltpu.SemaphoreType.DMA,
        pltpu.SemaphoreType.DMA,
        pltpu.SemaphoreType.DMA,
        pltpu.SemaphoreType.DMA,
    ],
)
def _sc_pool(edges_hbm, xs_lo_hbm, xs_hi_hbm, out_hbm,
             src_v, dst_v, buf_a, buf_b, work_v, cwork_v, ones_v,
             sums_sh, cnts_sh, sem_a, sem_b, sem_sa, sem_sb, sem_c):
    cid = lax.axis_index("c")
    sid = lax.axis_index("s")

    pltpu.sync_copy(edges_hbm.at[0, sid], src_v)
    pltpu.sync_copy(edges_hbm.at[1, sid], dst_v)

    def _zero_row(i, _):
        for j in range(DH // 16):
            work_v[i, pl.ds(j * 16, 16)] = jnp.zeros((16,), jnp.float32)
        cwork_v[i] = jnp.zeros((16,), jnp.float32)
        return 0

    lax.fori_loop(0, ZR, _zero_row, 0)

    def _one_row(i, _):
        ones_v[i] = jnp.full((16,), 1.0, jnp.float32)
        return 0

    lax.fori_loop(0, K, _one_row, 0)

    for t in range(RPT // ZR):
        row = sid * RPT + t * ZR
        pltpu.sync_copy(work_v, sums_sh.at[pl.ds(row, ZR)])
        pltpu.sync_copy(cwork_v, cnts_sh.at[pl.ds(row, ZR)])
    plsc.subcore_barrier()

    def _main_loop(xs_ref):
        def _gather(j, buf, sem):
            return pltpu.async_copy(xs_ref.at[src_v.at[j]], buf, sem)

        def _step(j, buf, gsem, ssem, nxt):
            pltpu.make_async_copy(xs_ref.at[src_v.at[j]], buf, gsem).wait()
            pltpu.sync_copy(buf, sums_sh.at[dst_v.at[j]], add=True)
            pltpu.sync_copy(ones_v, cnts_sh.at[dst_v.at[j]], add=True)

            @pl.when(nxt < NCHUNK)
            def _():
                _gather(nxt, buf, gsem)

        _gather(0, buf_a, sem_a)
        _gather(1, buf_b, sem_b)

        def _body(p, _):
            j = 2 * p
            _step(j, buf_a, sem_a, sem_sa, j + 2)
            _step(j + 1, buf_b, sem_b, sem_sb, j + 3)
            return 0

        lax.fori_loop(0, NCHUNK // 2, _body, 0)
        _step(NCHUNK - 1, buf_a, sem_a, sem_sa, NCHUNK)

    @pl.when(cid == 0)
    def _():
        _main_loop(xs_lo_hbm)

    @pl.when(cid == 1)
    def _():
        _main_loop(xs_hi_hbm)

    plsc.subcore_barrier()

    def _divide_and_store(row, nrows, col, t):
        pltpu.sync_copy(sums_sh.at[pl.ds(row, nrows)],
                        work_v.at[pl.ds(0, nrows)])
        pltpu.sync_copy(cnts_sh.at[pl.ds(row, nrows)],
                        cwork_v.at[pl.ds(0, nrows)])

        def _div_row(i, _):
            rec = 1.0 / jnp.maximum(cwork_v[i], 1.0)
            for j in range(DH // 16):
                work_v[i, pl.ds(j * 16, 16)] = (
                    work_v[i, pl.ds(j * 16, 16)] * rec)
            return 0

        lax.fori_loop(0, nrows, _div_row, 0)
        pltpu.sync_copy(work_v.at[pl.ds(0, nrows)],
                        out_hbm.at[pl.ds(row, nrows), pl.ds(col * DH, DH)])

    for col in range(NC):

        @pl.when(cid == col)
        def _():
            for t in range(RPT // ZR):
                row = sid * RPT + t * ZR

                @pl.when(row + ZR <= N)
                def _():
                    _divide_and_store(row, ZR, col, t)

                @pl.when(jnp.logical_and(row < N, row + ZR > N))
                def _():
                    _divide_and_store(row, TR, col, t)


@jax.jit
def kernel(xs, edge_index):
    src = edge_index[0].astype(jnp.int32).reshape(NS, EPW)
    dst = edge_index[1].astype(jnp.int32).reshape(NS, EPW)
    srcp = jnp.pad(src, ((0, 0), (0, PAD)))
    dstp = jnp.pad(dst, ((0, 0), (0, PAD)), constant_values=NP - 1)
    edges = jnp.stack([srcp, dstp]).reshape(2, NS, NCHUNK, K)
    xs2 = xs.reshape(N, 2, DH)
    return _sc_pool(edges, xs2[:, 0, :], xs2[:, 1, :])


# speedup vs baseline: 1.2156x; 1.2156x over previous
import functools

import jax
import jax.numpy as jnp
from jax import lax
from jax.experimental import pallas as pl
from jax.experimental.pallas import tpu as pltpu
from jax.experimental.pallas import tpu_sc as plsc

N = 10000
D = 128
DH = D // 2
E = 320000

NC = 2
NS = 16
EPW = E // NS
K = 125
NCHUNK = EPW // K
NP = 10240
RPT = NP // NS
ZR = 128
TR = N - (NS - 1) * RPT - 3 * ZR
CW = 16

_mesh = plsc.VectorSubcoreMesh(core_axis_name="c", subcore_axis_name="s")


@functools.partial(
    pl.kernel,
    out_type=jax.ShapeDtypeStruct((N, D), jnp.float32),
    mesh=_mesh,
    compiler_params=pltpu.CompilerParams(use_tc_tiling_on_sc=False),
    scratch_types=[
        pltpu.VMEM((NCHUNK, K), jnp.int32),
        pltpu.VMEM((NCHUNK, K), jnp.int32),
        pltpu.VMEM((K, DH), jnp.float32),
        pltpu.VMEM((K, DH), jnp.float32),
        pltpu.VMEM((ZR, DH), jnp.float32),
        pltpu.VMEM((ZR, CW), jnp.float32),
        pltpu.VMEM((K, CW), jnp.float32),
        pltpu.VMEM_SHARED((NP, DH), jnp.float32),
        pltpu.VMEM_SHARED((NP, CW), jnp.float32),
        pltpu.SemaphoreType.DMA,
        pltpu.SemaphoreType.DMA,
        pltpu.SemaphoreType.DMA,
        pltpu.SemaphoreType.DMA,
        pltpu.SemaphoreType.DMA,
    ],
)
def _sc_pool(edges_hbm, xs_lo_hbm, xs_hi_hbm, out_hbm,
             src_v, dst_v, buf_a, buf_b, work_v, cwork_v, ones_v,
             sums_sh, cnts_sh, sem_a, sem_b, sem_sa, sem_sb, sem_c):
    cid = lax.axis_index("c")
    sid = lax.axis_index("s")

    pltpu.sync_copy(edges_hbm.at[0, sid], src_v)
    pltpu.sync_copy(edges_hbm.at[1, sid], dst_v)

    def _zero_row(i, _):
        for j in range(DH // 16):
            work_v[i, pl.ds(j * 16, 16)] = jnp.zeros((16,), jnp.float32)
        cwork_v[i] = jnp.zeros((16,), jnp.float32)
        return 0

    lax.fori_loop(0, ZR, _zero_row, 0)

    def _one_row(i, _):
        ones_v[i] = jnp.full((16,), 1.0, jnp.float32)
        return 0

    lax.fori_loop(0, K, _one_row, 0)

    for t in range(RPT // ZR):
        row = sid * RPT + t * ZR
        pltpu.sync_copy(work_v, sums_sh.at[pl.ds(row, ZR)])
        pltpu.sync_copy(cwork_v, cnts_sh.at[pl.ds(row, ZR)])
    plsc.subcore_barrier()

    def _main_loop(xs_ref):
        def _gather(j, buf, sem):
            return pltpu.async_copy(xs_ref.at[src_v.at[j]], buf, sem)

        def _step(j, buf, gsem, ssem, nxt):
            pltpu.make_async_copy(xs_ref.at[src_v.at[j]], buf, gsem).wait()
            d1 = pltpu.async_copy(buf, sums_sh.at[dst_v.at[j]], ssem,
                                  add=True)
            d2 = pltpu.async_copy(ones_v, cnts_sh.at[dst_v.at[j]], sem_c,
                                  add=True)
            d1.wait()
            d2.wait()

            @pl.when(nxt < NCHUNK)
            def _():
                _gather(nxt, buf, gsem)

        _gather(0, buf_a, sem_a)
        _gather(1, buf_b, sem_b)

        def _body(p, _):
            j = 2 * p
            _step(j, buf_a, sem_a, sem_sa, j + 2)
            _step(j + 1, buf_b, sem_b, sem_sb, j + 3)
            return 0

        lax.fori_loop(0, NCHUNK // 2, _body, 0)

    @pl.when(cid == 0)
    def _():
        _main_loop(xs_lo_hbm)

    @pl.when(cid == 1)
    def _():
        _main_loop(xs_hi_hbm)

    plsc.subcore_barrier()

    def _divide_and_store(row, nrows, col, t):
        pltpu.sync_copy(sums_sh.at[pl.ds(row, nrows)],
                        work_v.at[pl.ds(0, nrows)])
        pltpu.sync_copy(cnts_sh.at[pl.ds(row, nrows)],
                        cwork_v.at[pl.ds(0, nrows)])

        def _div_row(i, _):
            rec = 1.0 / jnp.maximum(cwork_v[i], 1.0)
            for j in range(DH // 16):
                work_v[i, pl.ds(j * 16, 16)] = (
                    work_v[i, pl.ds(j * 16, 16)] * rec)
            return 0

        lax.fori_loop(0, nrows, _div_row, 0)
        pltpu.sync_copy(work_v.at[pl.ds(0, nrows)],
                        out_hbm.at[pl.ds(row, nrows), pl.ds(col * DH, DH)])

    for col in range(NC):

        @pl.when(cid == col)
        def _():
            for t in range(RPT // ZR):
                row = sid * RPT + t * ZR

                @pl.when(row + ZR <= N)
                def _():
                    _divide_and_store(row, ZR, col, t)

                @pl.when(jnp.logical_and(row < N, row + ZR > N))
                def _():
                    _divide_and_store(row, TR, col, t)


@jax.jit
def kernel(xs, edge_index):
    edges = edge_index.astype(jnp.int32).reshape(2, NS, NCHUNK, K)
    xs2 = xs.reshape(N, 2, DH)
    return _sc_pool(edges, xs2[:, 0, :], xs2[:, 1, :])
